# baseline (speedup 1.0000x reference)
import functools

import jax
import jax.numpy as jnp
from jax import lax
from jax.experimental import pallas as pl
from jax.experimental.pallas import tpu as pltpu
from jax.experimental.pallas import tpu_sc as plsc

NUM_EMB = 1000000
D = 64
NB = 16384
NT = 20
NC = 2
NS = 16
NW = NC * NS
BPW = NB // NW
CH = 256
CPT = BPW // CH
NCHUNK = NT * CPT


def _chunk_transpose(buf, bufT):
    iota = lax.iota(jnp.int32, 16)
    rows = [iota + 16 * g for g in range(CH // 16)]
    for d in range(D):
        col = jnp.full((16,), d, jnp.int32)
        for g in range(CH // 16):
            bufT[d, pl.ds(16 * g, 16)] = plsc.load_gather(buf, [rows[g], col])


def _emb_kernel(x_hbm, tab_hbm, out_hbm, idx_v, buf0, buf1, bufT0, bufT1,
                gsem0, gsem1, wsem0, wsem1):
    wid = lax.axis_index("s") * NC + lax.axis_index("c")
    b0 = wid * BPW
    pltpu.sync_copy(x_hbm.at[:, pl.ds(b0, BPW)], idx_v)

    bufs = (buf0, buf1)
    bufTs = (bufT0, bufT1)
    gsems = (gsem0, gsem1)
    wsems = (wsem0, wsem1)

    def gather(c, buf, sem):
        t = c // CPT
        hh = c % CPT
        for s in range(2):
            pltpu.async_copy(
                tab_hbm.at[idx_v.at[t, pl.ds(hh * CH + s * 128, 128)]],
                buf.at[pl.ds(s * 128, 128), :],
                sem,
            )

    def wait_gather(buf, sem):
        for _ in range(2):
            pltpu.make_async_copy(
                tab_hbm.at[idx_v.at[0, pl.ds(0, 128)]],
                buf.at[pl.ds(0, 128), :],
                sem,
            ).wait()

    def put(c, bufT, sem):
        t = c // CPT
        hh = c % CPT
        pltpu.async_copy(
            bufT, out_hbm.at[t, :, pl.ds(b0 + hh * CH, CH)], sem
        )

    def wait_put(bufT, sem):
        pltpu.make_async_copy(bufT, out_hbm.at[0, :, pl.ds(0, CH)], sem).wait()

    gather(0, buf0, gsem0)

    @pl.loop(0, NCHUNK // 2)
    def _(u):
        for p in range(2):
            c = 2 * u + p

            @pl.when(c + 1 < NCHUNK)
            def _():
                gather(c + 1, bufs[1 - p], gsems[1 - p])

            wait_gather(bufs[p], gsems[p])

            @pl.when(c >= 2)
            def _():
                wait_put(bufTs[p], wsems[p])

            _chunk_transpose(bufs[p], bufTs[p])
            put(c, bufTs[p], wsems[p])

    for p in range(2):
        wait_put(bufTs[p], wsems[p])


@jax.jit
def _emb(xt, weight):
    mesh = plsc.VectorSubcoreMesh(core_axis_name="c", subcore_axis_name="s")
    f = functools.partial(
        pl.kernel,
        mesh=mesh,
        out_type=jax.ShapeDtypeStruct((NT, D, NB), jnp.float32),
        scratch_types=[
            pltpu.VMEM((NT, BPW), jnp.int32),
            pltpu.VMEM((CH, D), jnp.float32),
            pltpu.VMEM((CH, D), jnp.float32),
            pltpu.VMEM((D, CH), jnp.float32),
            pltpu.VMEM((D, CH), jnp.float32),
            pltpu.SemaphoreType.DMA,
            pltpu.SemaphoreType.DMA,
            pltpu.SemaphoreType.DMA,
            pltpu.SemaphoreType.DMA,
        ],
        compiler_params=pltpu.CompilerParams(
            use_tc_tiling_on_sc=False, needs_layout_passes=False
        ),
    )(_emb_kernel)
    return f(xt, weight)


def kernel(x, weight):
    xt = x.astype(jnp.int32).T
    out = _emb(xt, weight)
    return jnp.transpose(out, (2, 0, 1))
